# baseline (speedup 1.0000x reference)
import functools

import jax
import jax.numpy as jnp
from jax import lax
from jax.experimental import pallas as pl
from jax.experimental.pallas import tpu as pltpu
from jax.experimental.pallas import tpu_sc as plsc

N = 10000
E = 320000
H = 128
G = 100

NP = 10112
ROWS_PER_TILE = NP // 16
NW = 32
CHUNK = 128
NCHUNK = 80
EPW = CHUNK * NCHUNK
EP = NW * EPW


def _hop_mesh():
    return plsc.VectorSubcoreMesh(core_axis_name="c", subcore_axis_name="s")


@functools.partial(
    pl.kernel,
    mesh=_hop_mesh(),
    out_type=jax.ShapeDtypeStruct((2, NP, H), jnp.float32),
    scratch_types=[
        pltpu.VMEM((NCHUNK // 2, CHUNK), jnp.int32),
        pltpu.VMEM((NCHUNK, CHUNK), jnp.int32),
        pltpu.VMEM((CHUNK, H), jnp.float32),
        pltpu.VMEM((CHUNK, H), jnp.float32),
        pltpu.VMEM_SHARED((NP, H), jnp.float32),
        pltpu.SemaphoreType.DMA,
        pltpu.SemaphoreType.DMA,
    ],
)
def _sc_hop(src_hbm, dst_hbm, hp_hbm, zeros_hbm, out_hbm,
            sidx_v, didx_v, rows_a, rows_b, acc_sh, sem_a, sem_b):
    c = lax.axis_index("c")
    s = lax.axis_index("s")
    w = s * 2 + c
    base = s * ROWS_PER_TILE
    NH = NCHUNK // 2
    pltpu.sync_copy(src_hbm.at[w].at[pl.ds(0, NH)], sidx_v)
    pltpu.sync_copy(dst_hbm.at[w], didx_v)
    pltpu.sync_copy(zeros_hbm, acc_sh.at[pl.ds(base, ROWS_PER_TILE)])
    plsc.subcore_barrier()

    pltpu.async_copy(hp_hbm.at[sidx_v.at[0]], rows_a, sem_a)
    pltpu.async_copy(hp_hbm.at[sidx_v.at[1]], rows_b, sem_b)

    def body(g, carry):
        j = 2 * g

        pltpu.make_async_copy(hp_hbm.at[pl.ds(0, CHUNK)], rows_a, sem_a).wait()
        pltpu.sync_copy(rows_a, acc_sh.at[didx_v.at[j]], add=True)
        pltpu.make_async_copy(hp_hbm.at[pl.ds(0, CHUNK)], rows_b, sem_b).wait()
        pltpu.sync_copy(rows_b, acc_sh.at[didx_v.at[j + 1]], add=True)

        @pl.when(j + 2 == NH)
        def _():
            pltpu.sync_copy(src_hbm.at[w].at[pl.ds(NH, NH)], sidx_v)

        @pl.when(g < NCHUNK // 2 - 1)
        def _():
            pltpu.async_copy(hp_hbm.at[sidx_v.at[(j + 2) % NH]], rows_a, sem_a)
            pltpu.async_copy(hp_hbm.at[sidx_v.at[(j + 3) % NH]], rows_b, sem_b)

        return carry

    lax.fori_loop(0, NCHUNK // 2, body, 0)
    plsc.subcore_barrier()
    pltpu.sync_copy(acc_sh.at[pl.ds(base, ROWS_PER_TILE)],
                    out_hbm.at[c].at[pl.ds(base, ROWS_PER_TILE)])


@functools.partial(
    pl.kernel,
    mesh=_hop_mesh(),
    out_type=jax.ShapeDtypeStruct((2, NP, H), jnp.float32),
    scratch_types=[
        pltpu.VMEM((NCHUNK, CHUNK), jnp.int32),
        pltpu.VMEM((CHUNK, H), jnp.float32),
        pltpu.VMEM_SHARED((NP, H), jnp.float32),
    ],
)
def _sc_deg(dst_hbm, ones_hbm, zeros_hbm, out_hbm, didx_v, ones_v, acc_sh):
    c = lax.axis_index("c")
    s = lax.axis_index("s")
    w = s * 2 + c
    base = s * ROWS_PER_TILE
    pltpu.sync_copy(dst_hbm.at[w], didx_v)
    pltpu.sync_copy(zeros_hbm, acc_sh.at[pl.ds(base, ROWS_PER_TILE)])
    pltpu.sync_copy(ones_hbm, ones_v)
    plsc.subcore_barrier()

    def body(j, carry):
        pltpu.sync_copy(ones_v, acc_sh.at[didx_v.at[j]], add=True)
        return carry

    lax.fori_loop(0, NCHUNK, body, 0)
    plsc.subcore_barrier()
    pltpu.sync_copy(acc_sh.at[pl.ds(base, ROWS_PER_TILE)],
                    out_hbm.at[c].at[pl.ds(base, ROWS_PER_TILE)])


def _head_body(xs_ref, mW_ref, mb_ref, f1W_ref, f1b_ref, f2W_ref, f2b_ref, out_ref):
    xs0 = xs_ref[:, 0, :]
    xs1 = xs_ref[:, 1, :]
    x_diff = jnp.abs(xs0 - xs1)
    x_mean = 0.5 * (xs0 + xs1)
    x_max = jnp.maximum(xs0, xs1)
    merged = (
        jnp.dot(x_diff, mW_ref[0:H, :], preferred_element_type=jnp.float32)
        + jnp.dot(x_mean, mW_ref[H:2 * H, :], preferred_element_type=jnp.float32)
        + jnp.dot(x_max, mW_ref[2 * H:3 * H, :], preferred_element_type=jnp.float32)
        + mb_ref[:]
    )
    f = jax.nn.relu(jnp.dot(merged, f1W_ref[:], preferred_element_type=jnp.float32) + f1b_ref[:])
    out_ref[:, :] = jnp.dot(f, f2W_ref[:], preferred_element_type=jnp.float32) + f2b_ref[:]


def _head(xs, merger_W, merger_b, ff1_W, ff1_b, ff2_W, ff2_b):
    Gp = 128
    xs_p = jnp.pad(xs, ((0, Gp - xs.shape[0]), (0, 0), (0, 0)))
    out = pl.pallas_call(
        _head_body,
        out_shape=jax.ShapeDtypeStruct((Gp, H), jnp.float32),
    )(xs_p, merger_W, merger_b, ff1_W, ff1_b, ff2_W, ff2_b)
    return out[: xs.shape[0]]


def kernel(x, edge_index, set_indices, batch_ids, num_graphs, W0, b0, W1, b1,
           ln0_g, ln0_b, ln1_g, ln1_b, merger_W, merger_b, ff1_W, ff1_b, ff2_W, ff2_b):
    src = edge_index[0]
    dst = edge_index[1]

    pad_e = EP - E
    src_p = jnp.concatenate([src, jnp.full((pad_e,), NP - 1, jnp.int32)]).reshape(NW, NCHUNK, CHUNK)
    dst_p = jnp.concatenate([dst, jnp.full((pad_e,), NP - 1, jnp.int32)]).reshape(NW, NCHUNK, CHUNK)
    x_p = jnp.pad(x, ((0, NP - N), (0, 0)))
    zeros_rows = jnp.zeros((ROWS_PER_TILE, H), jnp.float32)
    ones_rows = jnp.ones((CHUNK, H), jnp.float32)

    degp = _sc_deg(dst_p, ones_rows, zeros_rows)
    deg = degp[0, :, 0] + degp[1, :, 0] + 1.0
    dis = lax.rsqrt(deg)[:, None]

    def tag(h, W, b):
        out = h @ W[0]
        hp = dis * h
        for k in range(1, W.shape[0]):
            aggp = _sc_hop(src_p, dst_p, hp, zeros_rows)
            hk = dis * (aggp[0] + aggp[1]) + dis * hp
            out = out + hk @ W[k]
            hp = dis * hk
        return out + b

    def ln(h, g, b):
        mu = h.mean(axis=-1, keepdims=True)
        var = h.var(axis=-1, keepdims=True)
        return g * (h - mu) / jnp.sqrt(var + 1e-5) + b

    h = ln(jax.nn.relu(tag(x_p, W0, b0)), ln0_g, ln0_b)
    h = ln(jax.nn.relu(tag(h, W1, b1)), ln1_g, ln1_b)

    counts = jax.ops.segment_sum(jnp.ones((N,), jnp.float32), batch_ids, num_segments=G)
    counts = counts.astype(jnp.int32)
    index_bases = jnp.concatenate([jnp.zeros((1,), jnp.int32), jnp.cumsum(counts)[:-1].astype(jnp.int32)])
    sib = index_bases[:, None] + set_indices
    xs = h[sib]
    return _head(xs, merger_W, merger_b, ff1_W, ff1_b, ff2_W, ff2_b)


# speedup vs baseline: 3.3828x; 3.3828x over previous
import functools

import jax
import jax.numpy as jnp
from jax import lax
from jax.experimental import pallas as pl
from jax.experimental.pallas import tpu as pltpu
from jax.experimental.pallas import tpu_sc as plsc

N = 10000
E = 320000
H = 128
G = 100

NP = 10112
ROWS_PER_TILE = NP // 16
NW = 32
CHUNK = 128
NCHUNK = 80
EPW = CHUNK * NCHUNK
EP = NW * EPW


def _hop_mesh():
    return plsc.VectorSubcoreMesh(core_axis_name="c", subcore_axis_name="s")


@functools.partial(
    pl.kernel,
    mesh=_hop_mesh(),
    out_type=jax.ShapeDtypeStruct((2, NP, H), jnp.float32),
    scratch_types=[
        pltpu.VMEM((NCHUNK // 2, CHUNK), jnp.int32),
        pltpu.VMEM((NCHUNK, CHUNK), jnp.int32),
        pltpu.VMEM((CHUNK, H), jnp.float32),
        pltpu.VMEM((CHUNK, H), jnp.float32),
        pltpu.VMEM_SHARED((NP, H), jnp.float32),
        pltpu.SemaphoreType.DMA,
        pltpu.SemaphoreType.DMA,
    ],
)
def _sc_hop(src_hbm, dst_hbm, hp_hbm, zeros_hbm, out_hbm,
            sidx_v, didx_v, rows_a, rows_b, acc_sh, sem_a, sem_b):
    c = lax.axis_index("c")
    s = lax.axis_index("s")
    w = s * 2 + c
    base = s * ROWS_PER_TILE
    NH = NCHUNK // 2
    pltpu.sync_copy(src_hbm.at[w].at[pl.ds(0, NH)], sidx_v)
    pltpu.sync_copy(dst_hbm.at[w], didx_v)
    pltpu.sync_copy(zeros_hbm, acc_sh.at[pl.ds(base, ROWS_PER_TILE)])
    plsc.subcore_barrier()

    pltpu.async_copy(hp_hbm.at[sidx_v.at[0]], rows_a, sem_a)
    pltpu.async_copy(hp_hbm.at[sidx_v.at[1]], rows_b, sem_b)

    def body(g, carry):
        j = 2 * g

        pltpu.make_async_copy(hp_hbm.at[pl.ds(0, CHUNK)], rows_a, sem_a).wait()
        pltpu.sync_copy(rows_a, acc_sh.at[didx_v.at[j]], add=True)
        pltpu.make_async_copy(hp_hbm.at[pl.ds(0, CHUNK)], rows_b, sem_b).wait()
        pltpu.sync_copy(rows_b, acc_sh.at[didx_v.at[j + 1]], add=True)

        @pl.when(j + 2 == NH)
        def _():
            pltpu.sync_copy(src_hbm.at[w].at[pl.ds(NH, NH)], sidx_v)

        @pl.when(g < NCHUNK // 2 - 1)
        def _():
            pltpu.async_copy(hp_hbm.at[sidx_v.at[(j + 2) % NH]], rows_a, sem_a)
            pltpu.async_copy(hp_hbm.at[sidx_v.at[(j + 3) % NH]], rows_b, sem_b)

        return carry

    lax.fori_loop(0, NCHUNK // 2, body, 0)
    plsc.subcore_barrier()
    pltpu.sync_copy(acc_sh.at[pl.ds(base, ROWS_PER_TILE)],
                    out_hbm.at[c].at[pl.ds(base, ROWS_PER_TILE)])


@functools.partial(
    pl.kernel,
    mesh=_hop_mesh(),
    out_type=jax.ShapeDtypeStruct((2, NP, H), jnp.float32),
    scratch_types=[
        pltpu.VMEM((NCHUNK, CHUNK), jnp.int32),
        pltpu.VMEM((CHUNK, H), jnp.float32),
        pltpu.VMEM_SHARED((NP, H), jnp.float32),
    ],
)
def _sc_deg(dst_hbm, ones_hbm, zeros_hbm, out_hbm, didx_v, ones_v, acc_sh):
    c = lax.axis_index("c")
    s = lax.axis_index("s")
    w = s * 2 + c
    base = s * ROWS_PER_TILE
    pltpu.sync_copy(dst_hbm.at[w], didx_v)
    pltpu.sync_copy(zeros_hbm, acc_sh.at[pl.ds(base, ROWS_PER_TILE)])
    pltpu.sync_copy(ones_hbm, ones_v)
    plsc.subcore_barrier()

    def body(j, carry):
        pltpu.sync_copy(ones_v, acc_sh.at[didx_v.at[j]], add=True)
        return carry

    lax.fori_loop(0, NCHUNK, body, 0)
    plsc.subcore_barrier()
    pltpu.sync_copy(acc_sh.at[pl.ds(base, ROWS_PER_TILE)],
                    out_hbm.at[c].at[pl.ds(base, ROWS_PER_TILE)])


def _head_body(xs_ref, mW_ref, mb_ref, f1W_ref, f1b_ref, f2W_ref, f2b_ref, out_ref):
    xs0 = xs_ref[:, 0, :]
    xs1 = xs_ref[:, 1, :]
    x_diff = jnp.abs(xs0 - xs1)
    x_mean = 0.5 * (xs0 + xs1)
    x_max = jnp.maximum(xs0, xs1)
    merged = (
        jnp.dot(x_diff, mW_ref[0:H, :], preferred_element_type=jnp.float32)
        + jnp.dot(x_mean, mW_ref[H:2 * H, :], preferred_element_type=jnp.float32)
        + jnp.dot(x_max, mW_ref[2 * H:3 * H, :], preferred_element_type=jnp.float32)
        + mb_ref[:]
    )
    f = jax.nn.relu(jnp.dot(merged, f1W_ref[:], preferred_element_type=jnp.float32) + f1b_ref[:])
    out_ref[:, :] = jnp.dot(f, f2W_ref[:], preferred_element_type=jnp.float32) + f2b_ref[:]


def _head(xs, merger_W, merger_b, ff1_W, ff1_b, ff2_W, ff2_b):
    Gp = 128
    xs_p = jnp.pad(xs, ((0, Gp - xs.shape[0]), (0, 0), (0, 0)))
    out = pl.pallas_call(
        _head_body,
        out_shape=jax.ShapeDtypeStruct((Gp, H), jnp.float32),
    )(xs_p, merger_W, merger_b, ff1_W, ff1_b, ff2_W, ff2_b)
    return out[: xs.shape[0]]


def kernel(x, edge_index, set_indices, batch_ids, num_graphs, W0, b0, W1, b1,
           ln0_g, ln0_b, ln1_g, ln1_b, merger_W, merger_b, ff1_W, ff1_b, ff2_W, ff2_b):
    src = edge_index[0]
    dst = edge_index[1]

    pad_e = EP - E
    pad_idx = N + (jnp.arange(pad_e, dtype=jnp.int32) % (NP - N))
    src_p = jnp.concatenate([src, pad_idx]).reshape(NW, NCHUNK, CHUNK)
    dst_p = jnp.concatenate([dst, pad_idx]).reshape(NW, NCHUNK, CHUNK)
    x_p = jnp.pad(x, ((0, NP - N), (0, 0)))
    zeros_rows = jnp.zeros((ROWS_PER_TILE, H), jnp.float32)
    ones_rows = jnp.ones((CHUNK, H), jnp.float32)

    degp = _sc_deg(dst_p, ones_rows, zeros_rows)
    deg = degp[0, :, 0] + degp[1, :, 0] + 1.0
    dis = lax.rsqrt(deg)[:, None]

    def tag(h, W, b):
        out = h @ W[0]
        hp = dis * h
        for k in range(1, W.shape[0]):
            aggp = _sc_hop(src_p, dst_p, hp, zeros_rows)
            hk = dis * (aggp[0] + aggp[1]) + dis * hp
            out = out + hk @ W[k]
            hp = dis * hk
        return out + b

    def ln(h, g, b):
        mu = h.mean(axis=-1, keepdims=True)
        var = h.var(axis=-1, keepdims=True)
        return g * (h - mu) / jnp.sqrt(var + 1e-5) + b

    h = ln(jax.nn.relu(tag(x_p, W0, b0)), ln0_g, ln0_b)
    h = ln(jax.nn.relu(tag(h, W1, b1)), ln1_g, ln1_b)

    counts = jax.ops.segment_sum(jnp.ones((N,), jnp.float32), batch_ids, num_segments=G)
    counts = counts.astype(jnp.int32)
    index_bases = jnp.concatenate([jnp.zeros((1,), jnp.int32), jnp.cumsum(counts)[:-1].astype(jnp.int32)])
    sib = index_bases[:, None] + set_indices
    xs = h[sib]
    return _head(xs, merger_W, merger_b, ff1_W, ff1_b, ff2_W, ff2_b)
